# baseline (speedup 1.0000x reference)
import functools

import jax
import jax.numpy as jnp
from jax import lax
from jax.experimental import pallas as pl
from jax.experimental.pallas import tpu as pltpu
from jax.experimental.pallas import tpu_sc as plsc

_B, _S, _D = 4, 2048, 768
_NC, _NS = 2, 16
_NW = _NC * _NS
_PPW = _S // _NW
_CH = 32
_NCHUNK = _B * _PPW // _CH
_NBUF = 3
_LANES = 16

_mesh = plsc.VectorSubcoreMesh(core_axis_name="c", subcore_axis_name="s")


@functools.partial(
    pl.kernel,
    mesh=_mesh,
    out_type=jax.ShapeDtypeStruct((_B, _S, _D), jnp.float32),
    scratch_types=[
        pltpu.VMEM((_B, _PPW), jnp.int32),
        pltpu.VMEM((_PPW, _D), jnp.float32),
        pltpu.VMEM((_NBUF, _CH, _D), jnp.float32),
        pltpu.SemaphoreType.DMA,
        pltpu.SemaphoreType.DMA,
        pltpu.SemaphoreType.DMA,
        pltpu.SemaphoreType.DMA,
        pltpu.SemaphoreType.DMA,
        pltpu.SemaphoreType.DMA,
        pltpu.SemaphoreType.DMA,
    ],
)
def _emb_kernel(tok_hbm, ids_hbm, pos_hbm, out_hbm, idx_v, pos_v, tok_v,
                isem, gsem0, gsem1, gsem2, ssem0, ssem1, ssem2):
    wid = lax.axis_index("s") * _NC + lax.axis_index("c")
    p0 = wid * _PPW
    gsem = (gsem0, gsem1, gsem2)
    ssem = (ssem0, ssem1, ssem2)

    idx_copies = [
        pltpu.async_copy(ids_hbm.at[b, pl.ds(p0, _PPW)], idx_v.at[b], isem)
        for b in range(_B)
    ]
    for h in idx_copies:
        h.wait()

    gh = [None] * _NCHUNK
    sh = [None] * _NCHUNK

    def start_gather(i):
        b, k = divmod(i, _PPW // _CH)
        buf = i % _NBUF
        gh[i] = pltpu.async_copy(
            tok_hbm.at[idx_v.at[b, pl.ds(k * _CH, _CH)]],
            tok_v.at[buf], gsem[buf])

    start_gather(0)
    start_gather(1)
    pltpu.sync_copy(pos_hbm.at[pl.ds(p0, _PPW)], pos_v)
    for i in range(_NCHUNK):
        buf = i % _NBUF
        if i + 2 < _NCHUNK:
            if i - 1 >= 0:
                sh[i - 1].wait()
            start_gather(i + 2)
        gh[i].wait()
        b, k = divmod(i, _PPW // _CH)

        @plsc.parallel_loop(0, _CH, unroll=3)
        def _row(j, k=k, buf=buf):
            for cc in range(0, _D, _LANES):
                plsc.addupdate(
                    tok_v.at[buf, j, pl.ds(cc, _LANES)],
                    pos_v[k * _CH + j, pl.ds(cc, _LANES)])

        sh[i] = pltpu.async_copy(
            tok_v.at[buf], out_hbm.at[b, pl.ds(p0 + k * _CH, _CH)], ssem[buf])
    sh[_NCHUNK - 2].wait()
    sh[_NCHUNK - 1].wait()


def kernel(inputs, W_tok, W_pos):
    return _emb_kernel(W_tok, inputs.astype(jnp.int32), W_pos)


# speedup vs baseline: 1.1516x; 1.1516x over previous
import functools

import jax
import jax.numpy as jnp
from jax import lax
from jax.experimental import pallas as pl
from jax.experimental.pallas import tpu as pltpu
from jax.experimental.pallas import tpu_sc as plsc

_B, _S, _D = 4, 2048, 768
_NC, _NS = 2, 16
_NW = _NC * _NS
_PPW = _S // _NW
_CH = 32
_NCHUNK = _B * _PPW // _CH
_NBUF = 3
_LANES = 16

_mesh = plsc.VectorSubcoreMesh(core_axis_name="c", subcore_axis_name="s")


@functools.partial(
    pl.kernel,
    mesh=_mesh,
    out_type=jax.ShapeDtypeStruct((_B, _S, _D), jnp.float32),
    scratch_types=[
        pltpu.VMEM((_B, _PPW), jnp.int32),
        pltpu.VMEM((_PPW, _D), jnp.float32),
        pltpu.VMEM((_NBUF, _CH, _D), jnp.float32),
        pltpu.SemaphoreType.DMA,
        pltpu.SemaphoreType.DMA,
        pltpu.SemaphoreType.DMA,
        pltpu.SemaphoreType.DMA,
        pltpu.SemaphoreType.DMA,
        pltpu.SemaphoreType.DMA,
        pltpu.SemaphoreType.DMA,
    ],
)
def _emb_kernel(tok_hbm, ids_hbm, pos_hbm, out_hbm, idx_v, pos_v, tok_v,
                isem, gsem0, gsem1, gsem2, ssem0, ssem1, ssem2):
    wid = lax.axis_index("s") * _NC + lax.axis_index("c")
    p0 = wid * _PPW
    gsem = (gsem0, gsem1, gsem2)
    ssem = (ssem0, ssem1, ssem2)

    idx_copies = [
        pltpu.async_copy(ids_hbm.at[b, pl.ds(p0, _PPW)], idx_v.at[b], isem)
        for b in range(_B)
    ]
    for h in idx_copies:
        h.wait()

    gh = [None] * _NCHUNK
    sh = [None] * _NCHUNK

    def start_gather(i):
        b, k = divmod(i, _PPW // _CH)
        buf = i % _NBUF
        gh[i] = pltpu.async_copy(
            tok_hbm.at[idx_v.at[b, pl.ds(k * _CH, _CH)]],
            tok_v.at[buf], gsem[buf])

    start_gather(0)
    start_gather(1)
    pltpu.sync_copy(pos_hbm.at[pl.ds(p0, _PPW)], pos_v)
    for i in range(_NCHUNK):
        buf = i % _NBUF
        if i + 2 < _NCHUNK:
            if i - 1 >= 0:
                sh[i - 1].wait()
            start_gather(i + 2)
        with jax.named_scope("gwait"):
            gh[i].wait()
        b, k = divmod(i, _PPW // _CH)

        with jax.named_scope("posadd"):
            @plsc.parallel_loop(0, _CH, unroll=2)
            def _row(j, k=k, buf=buf):
                for cc in range(0, _D, _LANES):
                    plsc.addupdate(
                        tok_v.at[buf, j, pl.ds(cc, _LANES)],
                        pos_v[k * _CH + j, pl.ds(cc, _LANES)])

        sh[i] = pltpu.async_copy(
            tok_v.at[buf], out_hbm.at[b, pl.ds(p0 + k * _CH, _CH)], ssem[buf])
    sh[_NCHUNK - 2].wait()
    sh[_NCHUNK - 1].wait()


def kernel(inputs, W_tok, W_pos):
    return _emb_kernel(W_tok, inputs.astype(jnp.int32), W_pos)


# speedup vs baseline: 1.2020x; 1.0438x over previous
import functools

import jax
import jax.numpy as jnp
from jax import lax
from jax.experimental import pallas as pl
from jax.experimental.pallas import tpu as pltpu
from jax.experimental.pallas import tpu_sc as plsc

_B, _S, _D = 4, 2048, 768
_NC, _NS = 2, 16
_NW = _NC * _NS
_PPW = _S // _NW
_CH = 32
_NCHUNK = _B * _PPW // _CH
_NBUF = 3
_LANES = 16

_mesh = plsc.VectorSubcoreMesh(core_axis_name="c", subcore_axis_name="s")


@functools.partial(
    pl.kernel,
    mesh=_mesh,
    out_type=jax.ShapeDtypeStruct((_B, _S, _D), jnp.float32),
    scratch_types=[
        pltpu.VMEM((_B, _PPW), jnp.int32),
        pltpu.VMEM((_PPW, _D), jnp.float32),
        pltpu.VMEM((_NBUF, _CH, _D), jnp.float32),
        pltpu.SemaphoreType.DMA,
        pltpu.SemaphoreType.DMA,
        pltpu.SemaphoreType.DMA,
        pltpu.SemaphoreType.DMA,
        pltpu.SemaphoreType.DMA,
        pltpu.SemaphoreType.DMA,
        pltpu.SemaphoreType.DMA,
        pltpu.SemaphoreType.DMA,
    ],
)
def _emb_kernel(tok_hbm, ids_hbm, pos_hbm, out_hbm, idx_v, pos_v, tok_v,
                isem, psem, gsem0, gsem1, gsem2, ssem0, ssem1, ssem2):
    wid = lax.axis_index("s") * _NC + lax.axis_index("c")
    p0 = wid * _PPW
    gsem = (gsem0, gsem1, gsem2)
    ssem = (ssem0, ssem1, ssem2)

    idx_copies = [
        pltpu.async_copy(ids_hbm.at[b, pl.ds(p0, _PPW)], idx_v.at[b], isem)
        for b in range(_B)
    ]
    pos_copy = pltpu.async_copy(pos_hbm.at[pl.ds(p0, _PPW)], pos_v, psem)
    for h in idx_copies:
        h.wait()

    gh = [None] * _NCHUNK
    sh = [None] * _NCHUNK

    def start_gather(i):
        b, k = divmod(i, _PPW // _CH)
        buf = i % _NBUF
        gh[i] = pltpu.async_copy(
            tok_hbm.at[idx_v.at[b, pl.ds(k * _CH, _CH)]],
            tok_v.at[buf], gsem[buf])

    start_gather(0)
    start_gather(1)
    pos_copy.wait()
    for i in range(_NCHUNK):
        buf = i % _NBUF
        gh[i].wait()
        b, k = divmod(i, _PPW // _CH)

        @plsc.parallel_loop(0, _CH, unroll=2)
        def _row(j, k=k, buf=buf):
            for cc in range(0, _D, _LANES):
                plsc.addupdate(
                    tok_v.at[buf, j, pl.ds(cc, _LANES)],
                    pos_v[k * _CH + j, pl.ds(cc, _LANES)])

        sh[i] = pltpu.async_copy(
            tok_v.at[buf], out_hbm.at[b, pl.ds(p0 + k * _CH, _CH)], ssem[buf])
        if i + 2 < _NCHUNK:
            if i - 1 >= 0:
                sh[i - 1].wait()
            start_gather(i + 2)
    sh[_NCHUNK - 3].wait()
    sh[_NCHUNK - 2].wait()
    sh[_NCHUNK - 1].wait()


def kernel(inputs, W_tok, W_pos):
    return _emb_kernel(W_tok, inputs.astype(jnp.int32), W_pos)
